# baseline (speedup 1.0000x reference)
import jax
import jax.numpy as jnp
from jax import lax
from jax.experimental import pallas as pl
from jax.experimental.pallas import tpu as pltpu
from jax.experimental.pallas import tpu_sc as plsc

_E = 16
_L = 16
_NC, _NS = 2, 16
_NW = _NC * _NS


def _gate_body(rows_per_w):
    def body(x_hbm, out_hbm, col_v, out_v, sem):
        wid = lax.axis_index("s") * _NC + lax.axis_index("c")
        base = wid * rows_per_w
        cp = pltpu.async_copy(
            x_hbm.at[pl.ds(base, rows_per_w), pl.ds(0, _L)], col_v, sem)
        iota = lax.iota(jnp.int32, _L)
        zeros = jnp.zeros((_L,), jnp.float32)
        ones = jnp.ones((_L,), jnp.float32)
        col0 = jnp.zeros((_L,), jnp.int32)
        for j in range(rows_per_w * _E // _L):
            out_v[pl.ds(j * _L, _L)] = zeros
        cp.wait()
        for j in range(rows_per_w // _L):
            row_ids = j * _L + iota
            vals = plsc.load_gather(col_v, [row_ids, col0])
            pos = row_ids * _E + vals.astype(jnp.int32)
            plsc.store_scatter(out_v, [pos], ones)
        pltpu.sync_copy(
            out_v, out_hbm.at[pl.ds(base * _E, rows_per_w * _E)])
    return body


def kernel(x, garbage):
    B = x.shape[0]
    rows_per_w = B // _NW
    mesh = plsc.VectorSubcoreMesh(
        core_axis_name="c", subcore_axis_name="s",
        num_cores=_NC, num_subcores=_NS)
    out_flat = pl.kernel(
        _gate_body(rows_per_w),
        out_type=jax.ShapeDtypeStruct((B * _E,), jnp.float32),
        mesh=mesh,
        compiler_params=pltpu.CompilerParams(
            use_tc_tiling_on_sc=False, needs_layout_passes=False),
        scratch_types=[
            pltpu.VMEM((rows_per_w, _L), jnp.float32),
            pltpu.VMEM((rows_per_w * _E,), jnp.float32),
            pltpu.SemaphoreType.DMA,
        ],
    )(x)
    return out_flat.reshape(B, _E)


# speedup vs baseline: 1.9759x; 1.9759x over previous
import jax
import jax.numpy as jnp
from jax import lax
from jax.experimental import pallas as pl
from jax.experimental.pallas import tpu as pltpu
from jax.experimental.pallas import tpu_sc as plsc

_E = 16
_L = 16
_NC, _NS = 2, 16
_NW = _NC * _NS


def _gate_body(rows_per_w):
    def body(x_hbm, out_hbm, col_v, out_v, sem):
        wid = lax.axis_index("s") * _NC + lax.axis_index("c")
        base = wid * rows_per_w
        cp = pltpu.async_copy(
            x_hbm.at[pl.ds(base, rows_per_w), pl.ds(0, 128)], col_v, sem)
        iota = lax.iota(jnp.int32, _L)
        zeros = jnp.zeros((_L,), jnp.float32)
        ones = jnp.ones((_L,), jnp.float32)
        col0 = jnp.zeros((_L,), jnp.int32)
        for j in range(rows_per_w * _E // _L):
            out_v[pl.ds(j * _L, _L)] = zeros
        cp.wait()
        for j in range(rows_per_w // _L):
            row_ids = j * _L + iota
            vals = plsc.load_gather(col_v, [row_ids, col0])
            pos = row_ids * _E + vals.astype(jnp.int32)
            plsc.store_scatter(out_v, [pos], ones)
        pltpu.sync_copy(
            out_v, out_hbm.at[pl.ds(base * _E, rows_per_w * _E)])
    return body


def kernel(x, garbage):
    B = x.shape[0]
    rows_per_w = B // _NW
    mesh = plsc.VectorSubcoreMesh(
        core_axis_name="c", subcore_axis_name="s",
        num_cores=_NC, num_subcores=_NS)
    out_flat = pl.kernel(
        _gate_body(rows_per_w),
        out_type=jax.ShapeDtypeStruct((B * _E,), jnp.float32),
        mesh=mesh,
        compiler_params=pltpu.CompilerParams(needs_layout_passes=False),
        scratch_types=[
            pltpu.VMEM((rows_per_w, 128), jnp.float32),
            pltpu.VMEM((rows_per_w * _E,), jnp.float32),
            pltpu.SemaphoreType.DMA,
        ],
    )(x)
    return out_flat.reshape(B, _E)


# speedup vs baseline: 1.9944x; 1.0094x over previous
import jax
import jax.numpy as jnp
from jax import lax
from jax.experimental import pallas as pl
from jax.experimental.pallas import tpu as pltpu
from jax.experimental.pallas import tpu_sc as plsc

_E = 16
_L = 16
_NC, _NS = 2, 16
_NW = _NC * _NS


def _gate_body(rows_per_w):
    def body(x_hbm, out_hbm, col_v, out_v, sem):
        wid = lax.axis_index("s") * _NC + lax.axis_index("c")
        base = wid * rows_per_w
        cp = pltpu.async_copy(
            x_hbm.at[pl.ds(base, rows_per_w), pl.ds(0, 128)], col_v, sem)
        iota = lax.iota(jnp.int32, _L)
        zeros = jnp.zeros((_L,), jnp.float32)
        ones = jnp.ones((_L,), jnp.float32)
        col0 = jnp.zeros((_L,), jnp.int32)
        for j in range(rows_per_w):
            out_v[j, :] = zeros
        cp.wait()
        for j in range(rows_per_w // _L):
            row_ids = j * _L + iota
            vals = plsc.load_gather(col_v, [row_ids, col0])
            plsc.store_scatter(
                out_v, [row_ids, vals.astype(jnp.int32)], ones)
        pltpu.sync_copy(
            out_v, out_hbm.at[pl.ds(base, rows_per_w), :])
    return body


def kernel(x, garbage):
    B = x.shape[0]
    rows_per_w = B // _NW
    mesh = plsc.VectorSubcoreMesh(
        core_axis_name="c", subcore_axis_name="s",
        num_cores=_NC, num_subcores=_NS)
    return pl.kernel(
        _gate_body(rows_per_w),
        out_type=jax.ShapeDtypeStruct((B, _E), jnp.float32),
        mesh=mesh,
        compiler_params=pltpu.CompilerParams(needs_layout_passes=False),
        scratch_types=[
            pltpu.VMEM((rows_per_w, 128), jnp.float32),
            pltpu.VMEM((rows_per_w, _E), jnp.float32),
            pltpu.SemaphoreType.DMA,
        ],
    )(x)


# speedup vs baseline: 2.3790x; 1.1929x over previous
import jax
import jax.numpy as jnp
from jax import lax
from jax.experimental import pallas as pl
from jax.experimental.pallas import tpu as pltpu
from jax.experimental.pallas import tpu_sc as plsc

_E = 16
_L = 16
_NC, _NS = 2, 16
_NW = _NC * _NS


def _gate_body(rows_per_w):
    def body(x_hbm, out_hbm, col_v, out_v, sem):
        wid = lax.axis_index("s") * _NC + lax.axis_index("c")
        base = wid * rows_per_w
        cp = pltpu.async_copy(
            x_hbm.at[pl.ds(base, rows_per_w), pl.ds(0, 128)], col_v, sem)
        iota = lax.iota(jnp.int32, _L)
        zeros = jnp.zeros((_L,), jnp.float32)
        ones = jnp.ones((_L,), jnp.float32)
        col0 = jnp.zeros((_L,), jnp.int32)
        for e in range(_E):
            for j in range(rows_per_w // _L):
                out_v[e, pl.ds(j * _L, _L)] = zeros
        cp.wait()
        for j in range(rows_per_w // _L):
            tok = j * _L + iota
            vals = plsc.load_gather(col_v, [tok, col0])
            plsc.store_scatter(
                out_v, [vals.astype(jnp.int32), tok], ones)
        pltpu.sync_copy(out_v, out_hbm.at[:, pl.ds(base, rows_per_w)])
    return body


def kernel(x, garbage):
    B = x.shape[0]
    rows_per_w = B // _NW
    mesh = plsc.VectorSubcoreMesh(
        core_axis_name="c", subcore_axis_name="s",
        num_cores=_NC, num_subcores=_NS)
    out_t = pl.kernel(
        _gate_body(rows_per_w),
        out_type=jax.ShapeDtypeStruct((_E, B), jnp.float32),
        mesh=mesh,
        compiler_params=pltpu.CompilerParams(needs_layout_passes=False),
        scratch_types=[
            pltpu.VMEM((rows_per_w, 128), jnp.float32),
            pltpu.VMEM((_E, rows_per_w), jnp.float32),
            pltpu.SemaphoreType.DMA,
        ],
    )(x)
    return out_t.T
